# baseline (speedup 1.0000x reference)
import jax
import jax.numpy as jnp
from jax import lax
from jax.experimental import pallas as pl
from jax.experimental.pallas import tpu as pltpu
from jax.experimental.pallas import tpu_sc as plsc

_B, _N, _D = 4, 256, 256
_H = 1024
_ROWS = _B * _H
_BLK = 512

_SC_ROWS = 1536
_TC_STEPS = (_ROWS - _SC_ROWS) // _BLK
_SC_OFF = _SC_ROWS // _BLK

_NW = 32
_SC_RPW = _SC_ROWS // _NW
_CHR = 24
_NCH = _SC_RPW // _CHR


def _sc_body(d1o_hbm, d1_hbm, d2o_hbm, d2_hbm, out_hbm,
             a0, a1, b0, b1, accv, sems):
    wid = lax.axis_index("s") * 2 + lax.axis_index("c")
    r0 = wid * _SC_RPW
    abufs = (a0, a1)
    bbufs = (b0, b1)

    def chunk_sum(abuf, bbuf, accs):
        def step(c, accs):
            col = pl.ds(c * 16, 16)
            accs = list(accs)
            for r in range(_CHR):
                d = abuf[r, col] - bbuf[r, col]
                accs[r % 4] = accs[r % 4] + d * d
            return tuple(accs)
        return lax.fori_loop(0, _H // 16, step, accs)

    accs = tuple(jnp.zeros((16,), jnp.float32) for _ in range(4))
    pairs = ((d1o_hbm, d1_hbm), (d2o_hbm, d2_hbm))

    def start(j):
        p, i = divmod(j, _NCH)
        o_hbm, r_hbm = pairs[p]
        sl = pl.ds(r0 + i * _CHR, _CHR)
        k = j % 2
        ca = pltpu.async_copy(o_hbm.at[sl], abufs[k], sems.at[k])
        cb = pltpu.async_copy(r_hbm.at[sl], bbufs[k], sems.at[2 + k])
        return (ca, cb)

    chunks = list(range(2 * _NCH))
    copies = [None, None]
    copies[0] = start(0)
    for j in chunks:
        if j + 1 < len(chunks):
            copies[(j + 1) % 2] = start(j + 1)
        ca, cb = copies[j % 2]
        ca.wait()
        cb.wait()
        accs = chunk_sum(abufs[j % 2], bbufs[j % 2], accs)

    acc = (accs[0] + accs[1]) + (accs[2] + accs[3])
    accv[...] = acc
    pltpu.sync_copy(accv, out_hbm.at[wid])


def _sc_partials(d1o, d1, d2o, d2):
    mesh = plsc.VectorSubcoreMesh(core_axis_name="c", subcore_axis_name="s")
    return pl.kernel(
        _sc_body,
        out_type=jax.ShapeDtypeStruct((_NW, 16), jnp.float32),
        mesh=mesh,
        scratch_types=[
            pltpu.VMEM((_CHR, _H), jnp.float32),
            pltpu.VMEM((_CHR, _H), jnp.float32),
            pltpu.VMEM((_CHR, _H), jnp.float32),
            pltpu.VMEM((_CHR, _H), jnp.float32),
            pltpu.VMEM((16,), jnp.float32),
            pltpu.SemaphoreType.DMA((4,)),
        ],
    )(d1o, d1, d2o, d2)


def _tc_body(e1_ref, e2_ref, gt_ref, d1o_ref, d1_ref, d2o_ref, d2_ref,
             out_ref):
    g = pl.program_id(0)

    @pl.when(g == 0)
    def _():
        total = jnp.float32(0.0)
        count = jnp.float32(0.0)
        for b in range(_B):
            e1 = e1_ref[b]
            e2 = e2_ref[b]
            mask = (gt_ref[b] > 0).astype(jnp.float32)
            gram = lax.dot_general(e1, e2, (((1,), (1,)), ((), ())),
                                   preferred_element_type=jnp.float32)
            rc = jnp.sum(mask, axis=1)
            cc = jnp.sum(mask, axis=0)
            sqn1 = jnp.sum(e1 * e1, axis=1)
            sqn2 = jnp.sum(e2 * e2, axis=1)
            k = jnp.sum(mask)
            s = (jnp.sum(rc * sqn1) + jnp.sum(cc * sqn2)
                 - 2.0 * jnp.sum(mask * gram))
            mse = jnp.where(k > 0, s / jnp.maximum(k * jnp.float32(_D), 1.0),
                            jnp.float32(0.0))
            total = total + mse
            count = count + (k > 0).astype(jnp.float32)
        out_ref[0] = total
        out_ref[1] = count
        out_ref[2] = jnp.float32(0.0)

    blk = jnp.sum((d1o_ref[...] - d1_ref[...]) ** 2) \
        + jnp.sum((d2o_ref[...] - d2_ref[...]) ** 2)
    out_ref[2] += blk


def _tc_scalars(depth_emb1, depth_emb2, gt_matches, d1o, d1, d2o, d2):
    emb_spec = pl.BlockSpec((_B, _N, _D), lambda g: (0, 0, 0))
    gt_spec = pl.BlockSpec((_B, _N, _N), lambda g: (0, 0, 0))
    depth_spec = pl.BlockSpec((_BLK, _H), lambda g: (g + _SC_OFF, 0))
    return pl.pallas_call(
        _tc_body,
        grid=(_TC_STEPS,),
        in_specs=[emb_spec, emb_spec, gt_spec,
                  depth_spec, depth_spec, depth_spec, depth_spec],
        out_specs=pl.BlockSpec(memory_space=pltpu.SMEM),
        out_shape=jax.ShapeDtypeStruct((3,), jnp.float32),
    )(depth_emb1, depth_emb2, gt_matches, d1o, d1, d2o, d2)


def _combine_body(epoch_ref, tc_ref, sc_ref, out_ref):
    depth_sum = tc_ref[2] + jnp.sum(sc_ref[...])
    depth = depth_sum * jnp.float32(1.0 / (_H * _H))
    total = tc_ref[0] + jnp.where(epoch_ref[0] < 10, depth, jnp.float32(0.0))
    out_ref[0] = total / tc_ref[1]


def _combine(epoch_arr, tc_out, sc_out):
    return pl.pallas_call(
        _combine_body,
        in_specs=[pl.BlockSpec(memory_space=pltpu.SMEM),
                  pl.BlockSpec(memory_space=pltpu.SMEM),
                  pl.BlockSpec((_NW, 16), lambda: (0, 0))],
        out_specs=pl.BlockSpec(memory_space=pltpu.SMEM),
        out_shape=jax.ShapeDtypeStruct((1,), jnp.float32),
    )(epoch_arr, tc_out, sc_out)


def kernel(final_score, depth_emb1, depth_emb2, depth1_out, depth1,
           depth2_out, depth2, gt_matches, epoch):
    del final_score
    d1o = depth1_out.reshape(_ROWS, _H)
    d1 = depth1.reshape(_ROWS, _H)
    d2o = depth2_out.reshape(_ROWS, _H)
    d2 = depth2.reshape(_ROWS, _H)
    epoch_arr = jnp.asarray(epoch, jnp.int32).reshape(1)

    sc_out = _sc_partials(d1o, d1, d2o, d2)
    tc_out = _tc_scalars(depth_emb1, depth_emb2, gt_matches,
                         d1o, d1, d2o, d2)
    out = _combine(epoch_arr, tc_out, sc_out)
    return out.reshape(())


# speedup vs baseline: 1.6722x; 1.6722x over previous
import jax
import jax.numpy as jnp
from jax import lax
from jax.experimental import pallas as pl
from jax.experimental.pallas import tpu as pltpu

_B, _N, _D = 4, 256, 256
_H = 1024
_ROWS = _B * _H
_BLK = 512
_STEPS = _ROWS // _BLK


def _fused_body(epoch_ref, e1_ref, e2_ref, gt_ref,
                d1o_ref, d1_ref, d2o_ref, d2_ref,
                out_ref, acc_ref):
    g = pl.program_id(0)

    @pl.when(g == 0)
    def _():
        total = jnp.float32(0.0)
        count = jnp.float32(0.0)
        for b in range(_B):
            e1 = e1_ref[b]
            e2 = e2_ref[b]
            mask = (gt_ref[b] > 0).astype(jnp.float32)
            gram = lax.dot_general(e1, e2, (((1,), (1,)), ((), ())),
                                   preferred_element_type=jnp.float32)
            rc = jnp.sum(mask, axis=1)
            cc = jnp.sum(mask, axis=0)
            sqn1 = jnp.sum(e1 * e1, axis=1)
            sqn2 = jnp.sum(e2 * e2, axis=1)
            k = jnp.sum(mask)
            s = (jnp.sum(rc * sqn1) + jnp.sum(cc * sqn2)
                 - 2.0 * jnp.sum(mask * gram))
            mse = jnp.where(k > 0, s / jnp.maximum(k * jnp.float32(_D), 1.0),
                            jnp.float32(0.0))
            total = total + mse
            count = count + (k > 0).astype(jnp.float32)
        acc_ref[0] = total
        acc_ref[1] = count
        acc_ref[2] = jnp.float32(0.0)

    blk = jnp.sum((d1o_ref[...] - d1_ref[...]) ** 2) \
        + jnp.sum((d2o_ref[...] - d2_ref[...]) ** 2)
    acc_ref[2] += blk

    @pl.when(g == pl.num_programs(0) - 1)
    def _():
        depth = acc_ref[2] * jnp.float32(1.0 / (_H * _H))
        total = acc_ref[0] + jnp.where(epoch_ref[0] < 10, depth,
                                       jnp.float32(0.0))
        out_ref[0] = total / acc_ref[1]


def kernel(final_score, depth_emb1, depth_emb2, depth1_out, depth1,
           depth2_out, depth2, gt_matches, epoch):
    del final_score
    d1o = depth1_out.reshape(_ROWS, _H)
    d1 = depth1.reshape(_ROWS, _H)
    d2o = depth2_out.reshape(_ROWS, _H)
    d2 = depth2.reshape(_ROWS, _H)
    epoch_arr = jnp.asarray(epoch, jnp.int32).reshape(1)

    emb_spec = pl.BlockSpec((_B, _N, _D), lambda g: (0, 0, 0))
    gt_spec = pl.BlockSpec((_B, _N, _N), lambda g: (0, 0, 0))
    depth_spec = pl.BlockSpec((_BLK, _H), lambda g: (g, 0))

    out = pl.pallas_call(
        _fused_body,
        grid=(_STEPS,),
        in_specs=[
            pl.BlockSpec(memory_space=pltpu.SMEM),
            emb_spec, emb_spec, gt_spec,
            depth_spec, depth_spec, depth_spec, depth_spec,
        ],
        out_specs=pl.BlockSpec(memory_space=pltpu.SMEM),
        out_shape=jax.ShapeDtypeStruct((1,), jnp.float32),
        scratch_shapes=[pltpu.SMEM((3,), jnp.float32)],
    )(epoch_arr, depth_emb1, depth_emb2, gt_matches, d1o, d1, d2o, d2)
    return out.reshape(())
